# baseline (speedup 1.0000x reference)
import jax
import jax.numpy as jnp
from jax.experimental import pallas as pl
from jax.experimental.pallas import tpu as pltpu

_LABELNUM = 1000
_BLK = 2048
_CBLK = 128


def _onehot_block(idx_ref, out_ref):
    j = pl.program_id(1)
    idx = idx_ref[...]
    cols = jax.lax.broadcasted_iota(jnp.int32, out_ref.shape, 1) + j * _CBLK
    out_ref[...] = (cols == idx).astype(jnp.float32)


def kernel(input):
    B = input.shape[0]
    idx = input.astype(jnp.int32)
    return pl.pallas_call(
        _onehot_block,
        grid=(B // _BLK, pl.cdiv(_LABELNUM, _CBLK)),
        in_specs=[pl.BlockSpec((_BLK, 1), lambda i, j: (i, 0))],
        out_specs=pl.BlockSpec((_BLK, _CBLK), lambda i, j: (i, j)),
        out_shape=jax.ShapeDtypeStruct((B, _LABELNUM), jnp.float32),
        compiler_params=pltpu.CompilerParams(
            dimension_semantics=("parallel", "parallel"),
        ),
    )(idx)


# speedup vs baseline: 5.0872x; 5.0872x over previous
import jax
import jax.numpy as jnp
from jax.experimental import pallas as pl
from jax.experimental.pallas import tpu as pltpu

_LABELNUM = 1000
_BBLK = 2048


def _onehot_block(idx_ref, out_ref):
    idx = idx_ref[...]
    rows = jax.lax.broadcasted_iota(jnp.int32, out_ref.shape, 0)
    out_ref[...] = (rows == idx[0]).astype(jnp.float32)


def kernel(input):
    B = input.shape[0]
    nblk = B // _BBLK
    idx3 = input.astype(jnp.int32).reshape(nblk, 1, _BBLK)
    out_t = pl.pallas_call(
        _onehot_block,
        grid=(nblk,),
        in_specs=[pl.BlockSpec((1, 1, _BBLK), lambda i: (i, 0, 0))],
        out_specs=pl.BlockSpec((_LABELNUM, _BBLK), lambda i: (0, i)),
        out_shape=jax.ShapeDtypeStruct((_LABELNUM, B), jnp.float32),
        compiler_params=pltpu.CompilerParams(
            dimension_semantics=("parallel",),
        ),
    )(idx3)
    return out_t.T
